# baseline (speedup 1.0000x reference)
import functools

import jax
import jax.numpy as jnp
from jax import lax
from jax.experimental import pallas as pl
from jax.experimental.pallas import tpu as pltpu
from jax.experimental.pallas import tpu_sc as plsc

B = 128
V = 100000
LANES = 16
CH = 20000
NCHUNK = V // CH
VREGS = CH // LANES
NC = 2
NS = 16
NW = NC * NS
ROWS_PER_W = B // NW


def _rotate(buf, x, s):
    buf[pl.ds(0, LANES)] = x
    buf[pl.ds(LANES, LANES)] = x
    return buf[pl.ds(s, LANES)]


def _max_bcast(buf, v):
    for s in (8, 4, 2, 1):
        v = jnp.maximum(v, _rotate(buf, v, s))
    return v


def _argmax_bcast(fbuf, ibuf, v, i):
    for s in (8, 4, 2, 1):
        v2 = _rotate(fbuf, v, s)
        i2 = _rotate(ibuf, i, s)
        p = (v2 > v) | ((v2 == v) & (i2 < i))
        v = jnp.where(p, v2, v)
        i = jnp.where(p, i2, i)
    return v, i


def _sc_body(logits_hbm, temps_hbm, noise_hbm, out_hbm, lbuf, nbuf, tbuf, wbuf,
             rfbuf, ribuf):
    cid = lax.axis_index("c")
    sid = lax.axis_index("s")
    wid = sid * NC + cid

    win_vec = jnp.zeros((LANES,), jnp.int32)
    lane_iota = lax.iota(jnp.int32, LANES)

    for r in range(ROWS_PER_W):
        row = wid * ROWS_PER_W + r
        pltpu.sync_copy(temps_hbm.at[row], tbuf)
        t_vec = tbuf[...]

        def ph1_chunk(ci, macc):
            pltpu.sync_copy(logits_hbm.at[row, ci], lbuf)

            def ph1_v(k, acc):
                return jnp.maximum(acc, lbuf[pl.ds(k * LANES, LANES)])

            return lax.fori_loop(0, VREGS, ph1_v, macc, unroll=4)

        macc = lax.fori_loop(
            0, NCHUNK, ph1_chunk, jnp.full((LANES,), -jnp.inf, jnp.float32)
        )
        m_vec = _max_bcast(rfbuf, macc) / t_vec

        def ph2_chunk(ci, carry):
            pltpu.sync_copy(logits_hbm.at[row, ci], lbuf)
            pltpu.sync_copy(noise_hbm.at[row, ci], nbuf)
            base = ci * CH

            def ph2_v(k, c2):
                bv, bi = c2
                off = k * LANES
                tl = lbuf[pl.ds(off, LANES)] / t_vec
                e = jnp.exp(tl - m_vec)
                nc = jnp.maximum(nbuf[pl.ds(off, LANES)], jnp.float32(1e-10))
                u = e / nc
                idx = base + off + lane_iota
                p = u > bv
                return jnp.where(p, u, bv), jnp.where(p, idx, bi)

            return lax.fori_loop(0, VREGS, ph2_v, carry, unroll=4)

        bv, bi = lax.fori_loop(
            0,
            NCHUNK,
            ph2_chunk,
            (jnp.full((LANES,), -jnp.inf, jnp.float32), jnp.zeros((LANES,), jnp.int32)),
        )

        _, widx = _argmax_bcast(rfbuf, ribuf, bv, bi)
        win_vec = jnp.where(lane_iota == r, widx, win_vec)

    wbuf[...] = win_vec
    pltpu.sync_copy(wbuf, out_hbm.at[wid])


@jax.jit
def kernel(logits, temperatures, exp_noise):
    logits3 = logits.reshape(B, NCHUNK, CH)
    noise3 = exp_noise.reshape(B, NCHUNK, CH)
    temps2 = jnp.broadcast_to(temperatures[:, None], (B, LANES))
    mesh = plsc.VectorSubcoreMesh(core_axis_name="c", subcore_axis_name="s")
    sc = functools.partial(
        pl.kernel,
        mesh=mesh,
        out_type=jax.ShapeDtypeStruct((NW, LANES), jnp.int32),
        scratch_types=[
            pltpu.VMEM((CH,), jnp.float32),
            pltpu.VMEM((CH,), jnp.float32),
            pltpu.VMEM((LANES,), jnp.float32),
            pltpu.VMEM((LANES,), jnp.int32),
            pltpu.VMEM((2 * LANES,), jnp.float32),
            pltpu.VMEM((2 * LANES,), jnp.int32),
        ],
    )(_sc_body)
    out = sc(logits3, temps2, noise3)
    return out[:, :ROWS_PER_W].reshape(B)
